# baseline (speedup 1.0000x reference)
import functools

import jax
import jax.numpy as jnp
from jax import lax
from jax.experimental import pallas as pl
from jax.experimental.pallas import tpu as pltpu
from jax.experimental.pallas import tpu_sc as plsc

K = 1024
D = 32
N_ROWS = 8 * 56 * 56
ROW_TILE = 256

_NC, _NS = 2, 16
_NW = _NC * _NS
_BPW = N_ROWS // _NW
_CHUNK = 112
_NCHUNK = _BPW // _CHUNK


def _conv2d(x, w, b, stride, pad):
    y = lax.conv_general_dilated(x, w, (stride, stride), [(pad, pad), (pad, pad)],
                                 dimension_numbers=('NCHW', 'OIHW', 'NCHW'))
    return y + b[None, :, None, None]


def _conv_transpose2d(x, w, b, stride, pad):
    k = w.shape[2]
    w_c = jnp.flip(w, axis=(2, 3)).transpose(1, 0, 2, 3)
    p = k - 1 - pad
    y = lax.conv_general_dilated(x, w_c, (1, 1), [(p, p), (p, p)],
                                 lhs_dilation=(stride, stride),
                                 dimension_numbers=('NCHW', 'OIHW', 'NCHW'))
    return y + b[None, :, None, None]


def _vq_argmin_body(z_ref, cbt_ref, idx_ref, vqsum_ref):
    i = pl.program_id(0)
    z = z_ref[...]
    cbt = cbt_ref[...]
    prod = lax.dot_general(z, cbt, (((1,), (0,)), ((), ())),
                           preferred_element_type=jnp.float32)
    cbsq = jnp.sum(cbt * cbt, axis=0, keepdims=True)
    d2 = cbsq - 2.0 * prod
    minv = jnp.min(d2, axis=1, keepdims=True)
    kio = lax.broadcasted_iota(jnp.int32, d2.shape, 1)
    idx = jnp.min(jnp.where(d2 <= minv, kio, jnp.int32(K)), axis=1, keepdims=True)
    idx_ref[...] = idx
    zsq = jnp.sum(z * z, axis=1, keepdims=True)
    tile_sum = jnp.sum(jnp.maximum(minv + zsq, 0.0))

    @pl.when(i == 0)
    def _init():
        vqsum_ref[0, 0] = 0.0

    vqsum_ref[0, 0] += tile_sum


def _vq_argmin(z_flat, codebook_t):
    grid = N_ROWS // ROW_TILE
    return pl.pallas_call(
        _vq_argmin_body,
        grid=(grid,),
        in_specs=[
            pl.BlockSpec((ROW_TILE, D), lambda i: (i, 0)),
            pl.BlockSpec((D, K), lambda i: (0, 0)),
        ],
        out_specs=[
            pl.BlockSpec((ROW_TILE, 1), lambda i: (i, 0)),
            pl.BlockSpec(memory_space=pltpu.SMEM, block_shape=(1, 1),
                         index_map=lambda i: (0, 0)),
        ],
        out_shape=[
            jax.ShapeDtypeStruct((N_ROWS, 1), jnp.int32),
            jax.ShapeDtypeStruct((1, 1), jnp.float32),
        ],
        compiler_params=pltpu.CompilerParams(
            dimension_semantics=("arbitrary",)),
    )(z_flat, codebook_t)


@functools.cache
def _sc_gather_fn():
    @functools.partial(
        pl.kernel,
        mesh=plsc.VectorSubcoreMesh(core_axis_name="c", subcore_axis_name="s"),
        out_type=jax.ShapeDtypeStruct((N_ROWS, D), jnp.float32),
        scratch_types=[
            pltpu.VMEM((_BPW,), jnp.int32),
            pltpu.VMEM((_BPW, D), jnp.float32),
            pltpu.SemaphoreType.DMA,
        ],
        compiler_params=pltpu.CompilerParams(use_tc_tiling_on_sc=False),
    )
    def _sc_gather(table_hbm, idx_hbm, out_hbm, idx_v, rows_v, sem):
        wid = lax.axis_index("s") * _NC + lax.axis_index("c")
        base = wid * _BPW
        pltpu.sync_copy(idx_hbm.at[pl.ds(base, _BPW)], idx_v)
        cps = []
        for j in range(_NCHUNK):
            cps.append(pltpu.async_copy(
                table_hbm.at[idx_v.at[pl.ds(j * _CHUNK, _CHUNK)]],
                rows_v.at[pl.ds(j * _CHUNK, _CHUNK)], sem))
        for cp in cps:
            cp.wait()
        pltpu.sync_copy(rows_v, out_hbm.at[pl.ds(base, _BPW)])

    return _sc_gather


def kernel(x, enc_w1, enc_b1, enc_w2, enc_b2, enc_w3, enc_b3, codebook,
           dec_w1, dec_b1, dec_w2, dec_b2, dec_w3, dec_b3):
    h = jax.nn.relu(_conv2d(x, enc_w1, enc_b1, 2, 1))
    h = jax.nn.relu(_conv2d(h, enc_w2, enc_b2, 2, 1))
    z = _conv2d(h, enc_w3, enc_b3, 1, 1)
    B, Dz, H, W = z.shape
    z_flat = jnp.transpose(z, (0, 2, 3, 1)).reshape(-1, Dz)

    idx2, vq_sum = _vq_argmin(z_flat, codebook.T)
    vq_loss = vq_sum[0, 0] / (N_ROWS * D)

    z_q_flat = _sc_gather_fn()(codebook, idx2.reshape(-1))

    z_q = jnp.transpose(z_q_flat.reshape(B, H, W, Dz), (0, 3, 1, 2))
    z_q = z + lax.stop_gradient(z_q - z)
    d = jax.nn.relu(_conv_transpose2d(z_q, dec_w1, dec_b1, 1, 1))
    d = jax.nn.relu(_conv_transpose2d(d, dec_w2, dec_b2, 2, 1))
    x_recon = jax.nn.sigmoid(_conv_transpose2d(d, dec_w3, dec_b3, 2, 1))
    recon_loss = jnp.mean((x_recon - x) ** 2)
    return (x_recon, recon_loss, vq_loss)


# speedup vs baseline: 1.2478x; 1.2478x over previous
import functools

import jax
import jax.numpy as jnp
from jax import lax
from jax.experimental import pallas as pl
from jax.experimental.pallas import tpu as pltpu
from jax.experimental.pallas import tpu_sc as plsc

K = 1024
D = 32
N_ROWS = 8 * 56 * 56
ROW_TILE = 256

_NC, _NS = 2, 16
_NW = _NC * _NS
_BPW = N_ROWS // _NW


def _conv_nhwc(x, w_oihw, b, stride, pad):
    w = jnp.transpose(w_oihw, (2, 3, 1, 0))
    y = lax.conv_general_dilated(x, w, (stride, stride), [(pad, pad), (pad, pad)],
                                 dimension_numbers=('NHWC', 'HWIO', 'NHWC'))
    return y + b[None, None, None, :]


def _conv_transpose_nhwc(x, w_iohw, b, stride, pad):
    k = w_iohw.shape[2]
    w = jnp.transpose(jnp.flip(w_iohw, axis=(2, 3)), (2, 3, 0, 1))
    p = k - 1 - pad
    y = lax.conv_general_dilated(x, w, (1, 1), [(p, p), (p, p)],
                                 lhs_dilation=(stride, stride),
                                 dimension_numbers=('NHWC', 'HWIO', 'NHWC'))
    return y + b[None, None, None, :]


def _vq_argmin_body(z_ref, cbt_ref, idx_ref, vqsum_ref):
    i = pl.program_id(0)
    z = z_ref[...]
    cbt = cbt_ref[...]
    prod = lax.dot_general(z, cbt, (((1,), (0,)), ((), ())),
                           preferred_element_type=jnp.float32)
    cbsq = jnp.sum(cbt * cbt, axis=0, keepdims=True)
    d2 = cbsq - 2.0 * prod
    minv = jnp.min(d2, axis=1, keepdims=True)
    kio = lax.broadcasted_iota(jnp.int32, d2.shape, 1)
    idx = jnp.min(jnp.where(d2 <= minv, kio, jnp.int32(K)), axis=1, keepdims=True)
    idx_ref[...] = idx
    zsq = jnp.sum(z * z, axis=1, keepdims=True)
    tile_sum = jnp.sum(jnp.maximum(minv + zsq, 0.0))

    @pl.when(i == 0)
    def _init():
        vqsum_ref[0, 0] = 0.0

    vqsum_ref[0, 0] += tile_sum


def _vq_argmin(z_flat, codebook_t):
    grid = N_ROWS // ROW_TILE
    return pl.pallas_call(
        _vq_argmin_body,
        grid=(grid,),
        in_specs=[
            pl.BlockSpec((ROW_TILE, D), lambda i: (i, 0)),
            pl.BlockSpec((D, K), lambda i: (0, 0)),
        ],
        out_specs=[
            pl.BlockSpec((ROW_TILE, 1), lambda i: (i, 0)),
            pl.BlockSpec(memory_space=pltpu.SMEM, block_shape=(1, 1),
                         index_map=lambda i: (0, 0)),
        ],
        out_shape=[
            jax.ShapeDtypeStruct((N_ROWS, 1), jnp.int32),
            jax.ShapeDtypeStruct((1, 1), jnp.float32),
        ],
        compiler_params=pltpu.CompilerParams(
            dimension_semantics=("arbitrary",)),
    )(z_flat, codebook_t)


@functools.cache
def _sc_gather_fn():
    @functools.partial(
        pl.kernel,
        mesh=plsc.VectorSubcoreMesh(core_axis_name="c", subcore_axis_name="s"),
        out_type=jax.ShapeDtypeStruct((N_ROWS, D), jnp.float32),
        scratch_types=[
            pltpu.VMEM((_BPW,), jnp.int32),
            pltpu.VMEM((_BPW, D), jnp.float32),
            pltpu.SemaphoreType.DMA,
        ],
        compiler_params=pltpu.CompilerParams(use_tc_tiling_on_sc=False),
    )
    def _sc_gather(table_hbm, idx_hbm, out_hbm, idx_v, rows_v, sem):
        wid = lax.axis_index("s") * _NC + lax.axis_index("c")
        base = wid * _BPW
        pltpu.sync_copy(idx_hbm.at[pl.ds(base, _BPW)], idx_v)
        pltpu.async_copy(table_hbm.at[idx_v], rows_v, sem).wait()
        pltpu.sync_copy(rows_v, out_hbm.at[pl.ds(base, _BPW)])

    return _sc_gather


def kernel(x, enc_w1, enc_b1, enc_w2, enc_b2, enc_w3, enc_b3, codebook,
           dec_w1, dec_b1, dec_w2, dec_b2, dec_w3, dec_b3):
    xh = x.reshape(8, 1, 224, 224).transpose(0, 2, 3, 1)
    h = jax.nn.relu(_conv_nhwc(xh, enc_w1, enc_b1, 2, 1))
    h = jax.nn.relu(_conv_nhwc(h, enc_w2, enc_b2, 2, 1))
    z = _conv_nhwc(h, enc_w3, enc_b3, 1, 1)
    z_flat = z.reshape(-1, D)

    idx2, vq_sum = _vq_argmin(z_flat, codebook.T)
    vq_loss = vq_sum[0, 0] / (N_ROWS * D)

    z_q_flat = _sc_gather_fn()(codebook, idx2.reshape(-1))

    zq_st = (z_flat + lax.stop_gradient(z_q_flat - z_flat)).reshape(8, 56, 56, D)
    d = jax.nn.relu(_conv_transpose_nhwc(zq_st, dec_w1, dec_b1, 1, 1))
    d = jax.nn.relu(_conv_transpose_nhwc(d, dec_w2, dec_b2, 2, 1))
    xr = jax.nn.sigmoid(_conv_transpose_nhwc(d, dec_w3, dec_b3, 2, 1))
    recon_loss = jnp.mean((xr - xh) ** 2)
    x_recon = xr.transpose(0, 3, 1, 2)
    return (x_recon, recon_loss, vq_loss)
